# baseline (speedup 1.0000x reference)
import functools

import jax
import jax.numpy as jnp
from jax import lax
from jax.experimental import pallas as pl
from jax.experimental.pallas import tpu as pltpu
from jax.experimental.pallas import tpu_sc as plsc

D = 64
NW = 32
CHUNK = 64
NBUF = 2
BM_WORDS = 32768
V2C_PAD = 1048576


def _combine_body(fixed_ref, train_ref, out_ref):
    out_ref[...] = fixed_ref[...] + train_ref[...]


def _combine_tables(fixed, train):
    return pl.pallas_call(
        _combine_body,
        out_shape=jax.ShapeDtypeStruct(fixed.shape, fixed.dtype),
    )(fixed, train)


@functools.lru_cache(maxsize=None)
def _make_sc_lookup(n_tokens):
    npw = n_tokens // NW
    nchunk = npw // CHUNK
    assert nchunk % NBUF == 0
    ccap = npw // CHUNK
    mesh = plsc.VectorSubcoreMesh(core_axis_name="c", subcore_axis_name="s")

    @functools.partial(
        pl.kernel,
        out_type=jax.ShapeDtypeStruct((n_tokens, D), jnp.float32),
        mesh=mesh,
        scratch_types=[
            pltpu.VMEM((npw,), jnp.int32),
            pltpu.VMEM((npw,), jnp.float32),
            pltpu.VMEM((BM_WORDS,), jnp.int32),
            pltpu.VMEM((ccap, CHUNK), jnp.int32),
            pltpu.VMEM((ccap, CHUNK), jnp.int32),
            pltpu.VMEM((ccap, CHUNK), jnp.int32),
            pltpu.VMEM((NBUF, CHUNK, D), jnp.float32),
            pltpu.VMEM((NBUF, CHUNK, D), jnp.float32),
            pltpu.VMEM((CHUNK, D), jnp.float32),
            [pltpu.SemaphoreType.DMA] * NBUF,
            [pltpu.SemaphoreType.DMA] * NBUF,
        ],
        compiler_params=pltpu.CompilerParams(
            use_tc_tiling_on_sc=False, needs_layout_passes=False),
    )
    def sc_lookup(ids_hbm, vals_hbm, comb_hbm, reg_hbm, bm_hbm, v2c_hbm,
                  out_hbm,
                  ids_v, vals_v, bm_v, cid_v, crow_v, cmap_v,
                  reg_b, out_b, cust_b, sem_g, sem_w):
        wid = lax.axis_index("s") * 2 + lax.axis_index("c")
        base = wid * npw

        pltpu.sync_copy(bm_hbm, bm_v)
        pltpu.sync_copy(ids_hbm.at[pl.ds(base, npw)], ids_v)
        pltpu.sync_copy(vals_hbm.at[pl.ds(base, npw)], vals_v)

        iota16 = lax.iota(jnp.int32, 16)

        def mask_body(i, k):
            sl = pl.ds(i * 16, 16)
            tid = ids_v[sl]
            word = plsc.load_gather(bm_v, [tid >> 5])
            bit = (word >> (tid & 31)) & 1
            is_custom = bit > 0
            ids_v[sl] = jnp.where(is_custom, 0, tid)
            vals_v[sl] = jnp.where(is_custom, 0.0, vals_v[sl])
            pos = k + plsc.cumsum(bit) - 1
            row = pos >> 6
            col = pos & 63
            e_vec = i * 16 + iota16
            plsc.store_scatter(cid_v, [row, col], tid, mask=is_custom)
            plsc.store_scatter(crow_v, [row, col], base + e_vec, mask=is_custom)
            cnt = plsc.all_reduce_population_count(is_custom)
            return k + cnt[0]
        k_custom = lax.fori_loop(0, npw // 16, mask_body, jnp.int32(0))

        kpad = ((k_custom + CHUNK - 1) // CHUNK) * CHUNK

        @pl.when(k_custom > 0)
        def _():
            last = k_custom - 1
            lrow = last >> 6
            c16 = ((last & 63) >> 4) << 4
            vcid = cid_v[lrow, pl.ds(c16, 16)]
            vcrow = crow_v[lrow, pl.ds(c16, 16)]
            lane = last & 15
            sel = iota16 == lane
            neg16 = jnp.full((16,), -1, jnp.int32)
            cid_last = jnp.broadcast_to(
                lax.reduce_max(jnp.where(sel, vcid, neg16), (0,)), (16,))
            crow_last = jnp.broadcast_to(
                lax.reduce_max(jnp.where(sel, vcrow, neg16), (0,)), (16,))

            def pad_body(w, _):
                pos = k_custom + w * 16 + iota16
                msk = pos < kpad
                plsc.store_scatter(cid_v, [pos >> 6, pos & 63], cid_last,
                                   mask=msk)
                plsc.store_scatter(crow_v, [pos >> 6, pos & 63], crow_last,
                                   mask=msk)
                return 0
            lax.fori_loop(0, CHUNK // 16, pad_body, 0)

        def fire_gather(c, b):
            for q in range(CHUNK // 16):
                idx = ids_v[pl.ds(c * CHUNK + q * 16, 16)]
                pltpu.async_copy(reg_hbm.at[idx],
                                 reg_b.at[b].at[pl.ds(q * 16, 16)], sem_g[b])

        for b in range(NBUF):
            fire_gather(b, b)

        def pipe_body(i, _):
            for b in range(NBUF):
                c = i * NBUF + b
                co = c * CHUNK
                for q in range(CHUNK // 16):
                    pltpu.make_async_copy(
                        reg_hbm.at[ids_v[pl.ds(0, 16)]],
                        reg_b.at[b].at[pl.ds(q * 16, 16)], sem_g[b]).wait()

                @pl.when(c >= NBUF)
                def _():
                    pltpu.make_async_copy(
                        out_b.at[b], out_hbm.at[pl.ds(base, CHUNK)],
                        sem_w[b]).wait()

                def grp_body(g, _):
                    vv = vals_v[pl.ds(co + g * 16, 16)]
                    for j in range(16):
                        e = g * 16 + j
                        sp = jnp.broadcast_to(vv[j], (16,))
                        for d0 in range(0, D, 16):
                            dsl = pl.ds(d0, 16)
                            out_b[b, e, dsl] = reg_b[b, e, dsl] * sp
                    return 0
                lax.fori_loop(0, CHUNK // 16, grp_body, 0)

                pltpu.async_copy(
                    out_b.at[b], out_hbm.at[pl.ds(base + co, CHUNK)], sem_w[b])

                @pl.when(c + NBUF < nchunk)
                def _():
                    fire_gather(c + NBUF, b)
            return 0
        lax.fori_loop(0, nchunk // NBUF, pipe_body, 0)

        for b in range(NBUF):
            pltpu.make_async_copy(
                out_b.at[b], out_hbm.at[pl.ds(base, CHUNK)], sem_w[b]).wait()

        nbatch = (k_custom + CHUNK - 1) // CHUNK

        def cust_body(g, _):
            pltpu.sync_copy(v2c_hbm.at[cid_v.at[g]], cmap_v.at[g])
            pltpu.sync_copy(comb_hbm.at[cmap_v.at[g]], cust_b)
            pltpu.sync_copy(cust_b, out_hbm.at[crow_v.at[g]])
            return 0
        lax.fori_loop(0, nbatch, cust_body, 0)

    return sc_lookup


def kernel(feature_ids, feature_values, custom_fixed_table,
           custom_trainable_table, regular_table, vocab_to_custom):
    b, l = feature_ids.shape
    n = b * l
    ids = feature_ids.reshape(n)
    vals = feature_values.reshape(n)
    comb = _combine_tables(custom_fixed_table, custom_trainable_table)
    nw = (vocab_to_custom.shape[0] - 1) // 32
    bits = (vocab_to_custom[:nw * 32].reshape(nw, 32) > 0).astype(jnp.int32)
    bitmap = (bits << jnp.arange(32, dtype=jnp.int32)[None, :]).sum(
        axis=1, dtype=jnp.int32)
    bitmap = jnp.pad(bitmap, (0, BM_WORDS - nw))
    out = _make_sc_lookup(n)(ids, vals, comb, regular_table, bitmap,
                             vocab_to_custom)
    return out.reshape(b, l, D)


# speedup vs baseline: 1.0000x; 1.0000x over previous
import functools

import jax
import jax.numpy as jnp
from jax import lax
from jax.experimental import pallas as pl
from jax.experimental.pallas import tpu as pltpu
from jax.experimental.pallas import tpu_sc as plsc

D = 64
NW = 32
CHUNK = 64
NBUF = 2
BM_WORDS = 32768


def _combine_body(fixed_ref, train_ref, out_ref):
    out_ref[...] = fixed_ref[...] + train_ref[...]


def _combine_tables(fixed, train):
    return pl.pallas_call(
        _combine_body,
        out_shape=jax.ShapeDtypeStruct(fixed.shape, fixed.dtype),
    )(fixed, train)


@functools.lru_cache(maxsize=None)
def _make_sc_lookup(n_tokens):
    npw = n_tokens // NW
    nchunk = npw // CHUNK
    assert nchunk % NBUF == 0
    ccap = npw // CHUNK
    mesh = plsc.VectorSubcoreMesh(core_axis_name="c", subcore_axis_name="s")

    @functools.partial(
        pl.kernel,
        out_type=jax.ShapeDtypeStruct((n_tokens, D), jnp.float32),
        mesh=mesh,
        scratch_types=[
            pltpu.VMEM((npw,), jnp.int32),
            pltpu.VMEM((npw,), jnp.float32),
            pltpu.VMEM((BM_WORDS,), jnp.int32),
            pltpu.VMEM((ccap, CHUNK), jnp.int32),
            pltpu.VMEM((ccap, CHUNK), jnp.int32),
            pltpu.VMEM((ccap, CHUNK), jnp.int32),
            pltpu.VMEM((NBUF, CHUNK, D), jnp.float32),
            pltpu.VMEM((NBUF, CHUNK, D), jnp.float32),
            pltpu.VMEM((CHUNK, D), jnp.float32),
            [pltpu.SemaphoreType.DMA] * NBUF,
            [pltpu.SemaphoreType.DMA] * NBUF,
        ],
        compiler_params=pltpu.CompilerParams(
            use_tc_tiling_on_sc=False, needs_layout_passes=False),
    )
    def sc_lookup(ids_hbm, vals_hbm, comb_hbm, reg_hbm, bm_hbm, v2c_hbm,
                  out_hbm,
                  ids_v, vals_v, bm_v, cid_v, crow_v, cmap_v,
                  reg_b, out_b, cust_b, sem_g, sem_w):
        wid = lax.axis_index("s") * 2 + lax.axis_index("c")
        base = wid * npw

        pltpu.sync_copy(bm_hbm, bm_v)
        pltpu.sync_copy(ids_hbm.at[pl.ds(base, npw)], ids_v)
        pltpu.sync_copy(vals_hbm.at[pl.ds(base, npw)], vals_v)

        iota16 = lax.iota(jnp.int32, 16)

        def mask_body(i, k):
            sl = pl.ds(i * 16, 16)
            tid = ids_v[sl]
            word = plsc.load_gather(bm_v, [tid >> 5])
            bit = (word >> (tid & 31)) & 1
            is_custom = bit > 0
            ids_v[sl] = jnp.where(is_custom, 0, tid)
            vals_v[sl] = jnp.where(is_custom, 0.0, vals_v[sl])
            pos = k + plsc.cumsum(bit) - 1
            row = pos >> 6
            col = pos & 63
            e_vec = i * 16 + iota16
            plsc.store_scatter(cid_v, [row, col], tid, mask=is_custom)
            plsc.store_scatter(crow_v, [row, col], base + e_vec, mask=is_custom)
            cnt = plsc.all_reduce_population_count(is_custom)
            return k + cnt[0]
        k_custom = lax.fori_loop(0, npw // 16, mask_body, jnp.int32(0))

        kpad = ((k_custom + CHUNK - 1) // CHUNK) * CHUNK

        @pl.when(k_custom > 0)
        def _():
            last = k_custom - 1
            lrow = last >> 6
            c16 = ((last & 63) >> 4) << 4
            vcid = cid_v[lrow, pl.ds(c16, 16)]
            vcrow = crow_v[lrow, pl.ds(c16, 16)]
            lane = last & 15
            sel = iota16 == lane
            neg16 = jnp.full((16,), -1, jnp.int32)
            cid_last = jnp.broadcast_to(
                lax.reduce_max(jnp.where(sel, vcid, neg16), (0,)), (16,))
            crow_last = jnp.broadcast_to(
                lax.reduce_max(jnp.where(sel, vcrow, neg16), (0,)), (16,))

            def pad_body(w, _):
                pos = k_custom + w * 16 + iota16
                msk = pos < kpad
                plsc.store_scatter(cid_v, [pos >> 6, pos & 63], cid_last,
                                   mask=msk)
                plsc.store_scatter(crow_v, [pos >> 6, pos & 63], crow_last,
                                   mask=msk)
                return 0
            lax.fori_loop(0, CHUNK // 16, pad_body, 0)

        def fire_gather(c, b):
            for q in range(CHUNK // 16):
                idx = ids_v[pl.ds(c * CHUNK + q * 16, 16)]
                pltpu.async_copy(reg_hbm.at[idx],
                                 reg_b.at[b].at[pl.ds(q * 16, 16)], sem_g[b])

        for b in range(NBUF):
            fire_gather(b, b)

        def pipe_body(i, _):
            for b in range(NBUF):
                c = i * NBUF + b
                co = c * CHUNK
                for q in range(CHUNK // 16):
                    pltpu.make_async_copy(
                        reg_hbm.at[ids_v[pl.ds(0, 16)]],
                        reg_b.at[b].at[pl.ds(q * 16, 16)], sem_g[b]).wait()

                @pl.when(c >= NBUF)
                def _():
                    pltpu.make_async_copy(
                        out_b.at[b], out_hbm.at[pl.ds(base, CHUNK)],
                        sem_w[b]).wait()

                def grp_body(g, _):
                    vv = vals_v[pl.ds(co + g * 16, 16)]
                    for j in range(16):
                        e = g * 16 + j
                        sp = jnp.broadcast_to(vv[j], (16,))
                        for d0 in range(0, D, 16):
                            dsl = pl.ds(d0, 16)
                            out_b[b, e, dsl] = reg_b[b, e, dsl] * sp
                    return 0
                lax.fori_loop(0, CHUNK // 16, grp_body, 0)

                pltpu.async_copy(
                    out_b.at[b], out_hbm.at[pl.ds(base + co, CHUNK)], sem_w[b])

                @pl.when(c + NBUF < nchunk)
                def _():
                    fire_gather(c + NBUF, b)
            return 0
        lax.fori_loop(0, nchunk // NBUF, pipe_body, 0)

        for b in range(NBUF):
            pltpu.make_async_copy(
                out_b.at[b], out_hbm.at[pl.ds(base, CHUNK)], sem_w[b]).wait()

        nbatch = (k_custom + CHUNK - 1) // CHUNK

        def cust_body(g, _):
            pltpu.sync_copy(v2c_hbm.at[cid_v.at[g]], cmap_v.at[g])
            pltpu.sync_copy(comb_hbm.at[cmap_v.at[g]], cust_b)
            pltpu.sync_copy(cust_b, out_hbm.at[crow_v.at[g]])
            return 0
        lax.fori_loop(0, nbatch, cust_body, 0)

    return sc_lookup


def kernel(feature_ids, feature_values, custom_fixed_table,
           custom_trainable_table, regular_table, vocab_to_custom):
    b, l = feature_ids.shape
    n = b * l
    ids = feature_ids.reshape(n)
    vals = feature_values.reshape(n)
    comb = _combine_tables(custom_fixed_table, custom_trainable_table)
    nw = (vocab_to_custom.shape[0] - 1) // 32
    bits = (vocab_to_custom[:nw * 32].reshape(nw, 32) > 0).astype(jnp.int32)
    bitmap = (bits << jnp.arange(32, dtype=jnp.int32)[None, :]).sum(
        axis=1, dtype=jnp.int32)
    bitmap = jnp.pad(bitmap, (0, BM_WORDS - nw))
    out = _make_sc_lookup(n)(ids, vals, comb, regular_table, bitmap,
                             vocab_to_custom)
    return out.reshape(b, l, D)
